# baseline (speedup 1.0000x reference)
import functools

import jax
import jax.numpy as jnp
from jax import lax
from jax.experimental import pallas as pl
from jax.experimental.pallas import tpu as pltpu
from jax.experimental.pallas import tpu_sc as plsc

B, N, C, KNN = 8, 2048, 128, 16
M = B * N
R = 256
NW = 32
PW = M // NW
P = 8


def _knn_body(pts_ref, ptst_ref, out_ref):
    b = pl.program_id(0)
    r = pl.program_id(1)
    pt = pts_ref[0]
    ptt = ptst_ref[0]
    sq_i = jnp.sum(pt * pt, axis=1, keepdims=True)
    sq_j = jnp.sum(ptt * ptt, axis=0, keepdims=True)
    g = lax.dot_general(pt, ptt, (((1,), (0,)), ((), ())),
                        preferred_element_type=jnp.float32)
    d = sq_i + sq_j - 2.0 * g
    ii = lax.broadcasted_iota(jnp.int32, (R, N), 0) + r * R
    jj = lax.broadcasted_iota(jnp.int32, (R, N), 1)
    d = jnp.where(jj == ii, 1e10, d)
    kb = lax.bitcast_convert_type(d, jnp.int32)
    kb = (kb & jnp.int32(~(N - 1))) | jj
    kf = lax.bitcast_convert_type(kb, jnp.float32)
    cols = []
    for _ in range(KNN):
        m = jnp.min(kf, axis=1, keepdims=True)
        cols.append(lax.bitcast_convert_type(m, jnp.int32) & jnp.int32(N - 1))
        kf = jnp.where(kf == m, jnp.float32(3e38), kf)
    idx = jnp.concatenate(cols, axis=1)
    out_ref[...] = idx + b * N


def _knn_indices(points, pts_t, nb):
    return pl.pallas_call(
        _knn_body,
        grid=(nb, N // R),
        in_specs=[
            pl.BlockSpec((1, R, 3), lambda b, r: (b, r, 0)),
            pl.BlockSpec((1, 3, N), lambda b, r: (b, 0, 0)),
        ],
        out_specs=pl.BlockSpec((R, KNN), lambda b, r: (b * (N // R) + r, 0)),
        out_shape=jax.ShapeDtypeStruct((nb * N, KNN), jnp.int32),
    )(points, pts_t)


NBUF = 4


def _make_sc_agg(with_x, m):
    mesh = plsc.VectorSubcoreMesh(core_axis_name="c", subcore_axis_name="s")
    scale = -1.0 / KNN
    pw = m // NW
    nch = pw // P
    scratch = [pltpu.VMEM((nch, P * KNN), jnp.int32)]
    scratch += [pltpu.VMEM((P * KNN, C), jnp.float32)] * NBUF
    if with_x:
        scratch += [pltpu.VMEM((P, C), jnp.float32)] * NBUF
    scratch += [pltpu.VMEM((P, C), jnp.float32)] * NBUF
    scratch += [pltpu.SemaphoreType.DMA] * (NBUF * (3 if with_x else 2))

    def _body(table_hbm, gidx_hbm, out_hbm, idxs, *bufs):
        rows = bufs[:NBUF]
        k = NBUF
        if with_x:
            xc = bufs[k:k + NBUF]
            k += NBUF
        outb = bufs[k:k + NBUF]
        k += NBUF
        sg = bufs[k:k + NBUF]
        k += NBUF
        if with_x:
            sx = bufs[k:k + NBUF]
            k += NBUF
        so = bufs[k:k + NBUF]

        wid = lax.axis_index("s") * 2 + lax.axis_index("c")
        base = wid * pw

        def fire(c, b):
            pltpu.async_copy(table_hbm.at[idxs.at[c]], rows[b], sg[b])
            if with_x:
                pltpu.async_copy(table_hbm.at[pl.ds(base + c * P, P)],
                                 xc[b], sx[b])

        def wait_fire(c, b):
            pltpu.make_async_copy(table_hbm.at[idxs.at[c]], rows[b],
                                  sg[b]).wait()
            if with_x:
                pltpu.make_async_copy(table_hbm.at[pl.ds(base + c * P, P)],
                                      xc[b], sx[b]).wait()

        def put_out(c, b):
            pltpu.async_copy(outb[b], out_hbm.at[pl.ds(base + c * P, P)],
                             so[b])

        def wait_out(c, b):
            pltpu.make_async_copy(outb[b],
                                  out_hbm.at[pl.ds(base + c * P, P)],
                                  so[b]).wait()

        def accumulate(b):
            def pbody(p, carry):
                for ch in range(C // 16):
                    sl = pl.ds(ch * 16, 16)
                    s = [rows[b][p * KNN + rr, sl]
                         + rows[b][p * KNN + rr + 1, sl]
                         for rr in range(0, KNN, 2)]
                    while len(s) > 1:
                        s = [a + bb for a, bb in zip(s[::2], s[1::2])]
                    if with_x:
                        outb[b][p, sl] = xc[b][p, sl] + scale * s[0]
                    else:
                        outb[b][p, sl] = s[0]
                return carry
            lax.fori_loop(0, P, pbody, 0)

        pltpu.sync_copy(gidx_hbm.at[pl.ds(wid * nch, nch)], idxs)
        for b in range(NBUF):
            fire(b, b)

        def group(j0, carry):
            for b in range(NBUF):
                c = j0 * NBUF + b
                wait_fire(c, b)

                @pl.when(j0 > 0)
                def _():
                    wait_out(c - NBUF, b)
                accumulate(b)
                put_out(c, b)

                @pl.when(j0 < nch // NBUF - 1)
                def _():
                    fire(c + NBUF, b)
            return carry

        lax.fori_loop(0, nch // NBUF, group, 0)
        for b in range(NBUF):
            wait_out(nch - NBUF + b, b)

    return functools.partial(
        pl.kernel, mesh=mesh,
        out_type=jax.ShapeDtypeStruct((m, C), jnp.float32),
        scratch_types=scratch,
    )(_body)


def _layernorm(x, g, b):
    mu = jnp.mean(x, axis=-1, keepdims=True)
    var = jnp.mean((x - mu) * (x - mu), axis=-1, keepdims=True)
    return (x - mu) / jnp.sqrt(var + 1e-5) * g + b


def _dense_body(x_ref, t1_ref, s1_ref, wsp_ref, bsp_ref, thl_ref, bl_ref,
                thh_ref, bh_ref, wg1_ref, bg1_ref, g1g_ref, g1b_ref,
                wg2_ref, bg2_ref, wout_ref, bout_ref, og_ref, ob_ref,
                gr_ref, out_ref):
    x = x_ref[...]
    t1 = t1_ref[...]
    t2 = 2.0 * t1 - (2.0 / KNN) * s1_ref[...] - x

    def mm(a, w):
        return jnp.dot(a, w, preferred_element_type=jnp.float32)

    f_sp = mm(x, wsp_ref[...]) + bsp_ref[...]
    f_lo = mm(x, thl_ref[0]) + mm(t1, thl_ref[1]) + mm(t2, thl_ref[2]) + bl_ref[...]
    f_hi = mm(x, thh_ref[0]) + mm(t1, thh_ref[1]) + mm(t2, thh_ref[2]) + bh_ref[...]
    h = (mm(f_sp, wg1_ref[0]) + mm(f_lo, wg1_ref[1]) + mm(f_hi, wg1_ref[2])
         + bg1_ref[...])
    h = jax.nn.relu(_layernorm(h, g1g_ref[...], g1b_ref[...]))
    gate = mm(h, wg2_ref[...]) + bg2_ref[...]
    gate = gate - jnp.max(gate, axis=-1, keepdims=True)
    e = jnp.exp(gate)
    gate = e / jnp.sum(e, axis=-1, keepdims=True)
    f_fused = (gate[:, 0:1] * f_sp + gate[:, 1:2] * f_lo + gate[:, 2:3] * f_hi)
    out = mm(f_fused, wout_ref[...]) + bout_ref[...]
    out = _layernorm(out, og_ref[...], ob_ref[...])
    out_ref[...] = x + gr_ref[0, 0] * out


def _dense_tail(x, t1, s1, wsp, bsp, thl, bl, thh, bh, wg1, bg1, g1g, g1b,
                wg2, bg2, wout, bout, og, ob, gr):
    row = lambda t: (t, 0)
    full2 = lambda t: (0, 0)
    full3 = lambda t: (0, 0, 0)
    specs = [
        pl.BlockSpec((R, C), row),
        pl.BlockSpec((R, C), row),
        pl.BlockSpec((R, C), row),
        pl.BlockSpec((C, C), full2),
        pl.BlockSpec((1, C), full2),
        pl.BlockSpec((3, C, C), full3),
        pl.BlockSpec((1, C), full2),
        pl.BlockSpec((3, C, C), full3),
        pl.BlockSpec((1, C), full2),
        pl.BlockSpec((3, C, C), full3),
        pl.BlockSpec((1, C), full2),
        pl.BlockSpec((1, C), full2),
        pl.BlockSpec((1, C), full2),
        pl.BlockSpec((C, 3), full2),
        pl.BlockSpec((1, 3), full2),
        pl.BlockSpec((C, C), full2),
        pl.BlockSpec((1, C), full2),
        pl.BlockSpec((1, C), full2),
        pl.BlockSpec((1, C), full2),
        pl.BlockSpec((1, 1), full2),
    ]
    m = x.shape[0]
    return pl.pallas_call(
        _dense_body,
        grid=(m // R,),
        in_specs=specs,
        out_specs=pl.BlockSpec((R, C), row),
        out_shape=jax.ShapeDtypeStruct((m, C), jnp.float32),
    )(x, t1, s1, wsp, bsp, thl, bl, thh, bh, wg1, bg1, g1g, g1b,
      wg2, bg2, wout, bout, og, ob, gr)


NSPLIT = 8


@functools.lru_cache(maxsize=None)
def _sc_aggs(m):
    return _make_sc_agg(True, m), _make_sc_agg(False, m)


def kernel(points, feats, W_sp, b_sp, theta_low, b_low, theta_high, b_high,
           W_g1, b_g1, ln_g1_gamma, ln_g1_beta, W_g2, b_g2, W_out, b_out,
           ln_out_gamma, ln_out_beta, gamma_res):
    r2 = lambda v: v.reshape(1, -1)
    nb = B // NSPLIT
    mh = nb * N
    sc_l, sc_sum = _sc_aggs(mh)
    outs = []
    for h in range(NSPLIT):
        pts = points[h * nb:(h + 1) * nb]
        xf = feats[h * nb:(h + 1) * nb].reshape(mh, C)
        gidx = _knn_indices(pts, jnp.swapaxes(pts, 1, 2),
                            nb).reshape(mh // P, P * KNN)
        t1 = sc_l(xf, gidx)
        s1 = sc_sum(t1, gidx)
        out = _dense_tail(
            xf, t1, s1, W_sp, r2(b_sp), theta_low, r2(b_low), theta_high,
            r2(b_high), W_g1.reshape(3, C, C), r2(b_g1), r2(ln_g1_gamma),
            r2(ln_g1_beta), W_g2, r2(b_g2), W_out, r2(b_out),
            r2(ln_out_gamma), r2(ln_out_beta), gamma_res.reshape(1, 1))
        outs.append(out.reshape(nb, N, C))
    return jnp.concatenate(outs, axis=0)


# speedup vs baseline: 1.0728x; 1.0728x over previous
import functools

import jax
import jax.numpy as jnp
from jax import lax
from jax.experimental import pallas as pl
from jax.experimental.pallas import tpu as pltpu
from jax.experimental.pallas import tpu_sc as plsc

B, N, C, KNN = 8, 2048, 128, 16
M = B * N
R = 256
NW = 32
PW = M // NW
P = 8


def _knn_body(pts_ref, ptst_ref, out_ref):
    b = pl.program_id(0)
    r = pl.program_id(1)
    pt = pts_ref[0]
    ptt = ptst_ref[0]
    sq_i = jnp.sum(pt * pt, axis=1, keepdims=True)
    sq_j = jnp.sum(ptt * ptt, axis=0, keepdims=True)
    g = lax.dot_general(pt, ptt, (((1,), (0,)), ((), ())),
                        preferred_element_type=jnp.float32)
    d = sq_i + sq_j - 2.0 * g
    ii = lax.broadcasted_iota(jnp.int32, (R, N), 0) + r * R
    jj = lax.broadcasted_iota(jnp.int32, (R, N), 1)
    d = jnp.where(jj == ii, 1e10, d)
    kb = lax.bitcast_convert_type(d, jnp.int32)
    kb = (kb & jnp.int32(~(N - 1))) | jj
    kf = lax.bitcast_convert_type(kb, jnp.float32)
    cols = []
    for _ in range(KNN):
        m = jnp.min(kf, axis=1, keepdims=True)
        cols.append(lax.bitcast_convert_type(m, jnp.int32) & jnp.int32(N - 1))
        kf = jnp.where(kf == m, jnp.float32(3e38), kf)
    idx = jnp.concatenate(cols, axis=1)
    out_ref[...] = idx + b * N


def _knn_indices(points, pts_t, nb):
    return pl.pallas_call(
        _knn_body,
        grid=(nb, N // R),
        in_specs=[
            pl.BlockSpec((1, R, 3), lambda b, r: (b, r, 0)),
            pl.BlockSpec((1, 3, N), lambda b, r: (b, 0, 0)),
        ],
        out_specs=pl.BlockSpec((R, KNN), lambda b, r: (b * (N // R) + r, 0)),
        out_shape=jax.ShapeDtypeStruct((nb * N, KNN), jnp.int32),
    )(points, pts_t)


NBUF = 4


def _make_sc_agg(with_x, m):
    mesh = plsc.VectorSubcoreMesh(core_axis_name="c", subcore_axis_name="s")
    scale = -1.0 / KNN
    pw = m // NW
    nch = pw // P
    scratch = [pltpu.VMEM((nch, P * KNN), jnp.int32)]
    scratch += [pltpu.VMEM((P * KNN, C), jnp.float32)] * NBUF
    if with_x:
        scratch += [pltpu.VMEM((P, C), jnp.float32)] * NBUF
    scratch += [pltpu.VMEM((P, C), jnp.float32)] * NBUF
    scratch += [pltpu.SemaphoreType.DMA] * (NBUF * (3 if with_x else 2))

    def _body(table_hbm, gidx_hbm, out_hbm, idxs, *bufs):
        rows = bufs[:NBUF]
        k = NBUF
        if with_x:
            xc = bufs[k:k + NBUF]
            k += NBUF
        outb = bufs[k:k + NBUF]
        k += NBUF
        sg = bufs[k:k + NBUF]
        k += NBUF
        if with_x:
            sx = bufs[k:k + NBUF]
            k += NBUF
        so = bufs[k:k + NBUF]

        wid = lax.axis_index("s") * 2 + lax.axis_index("c")
        base = wid * pw

        def fire(c, b):
            pltpu.async_copy(table_hbm.at[idxs.at[c]], rows[b], sg[b])
            if with_x:
                pltpu.async_copy(table_hbm.at[pl.ds(base + c * P, P)],
                                 xc[b], sx[b])

        def wait_fire(c, b):
            pltpu.make_async_copy(table_hbm.at[idxs.at[c]], rows[b],
                                  sg[b]).wait()
            if with_x:
                pltpu.make_async_copy(table_hbm.at[pl.ds(base + c * P, P)],
                                      xc[b], sx[b]).wait()

        def put_out(c, b):
            pltpu.async_copy(outb[b], out_hbm.at[pl.ds(base + c * P, P)],
                             so[b])

        def wait_out(c, b):
            pltpu.make_async_copy(outb[b],
                                  out_hbm.at[pl.ds(base + c * P, P)],
                                  so[b]).wait()

        def accumulate(b):
            def pbody(p, carry):
                for ch in range(C // 16):
                    sl = pl.ds(ch * 16, 16)
                    s = [rows[b][p * KNN + rr, sl]
                         + rows[b][p * KNN + rr + 1, sl]
                         for rr in range(0, KNN, 2)]
                    while len(s) > 1:
                        s = [a + bb for a, bb in zip(s[::2], s[1::2])]
                    if with_x:
                        outb[b][p, sl] = xc[b][p, sl] + scale * s[0]
                    else:
                        outb[b][p, sl] = s[0]
                return carry
            lax.fori_loop(0, P, pbody, 0)

        pltpu.sync_copy(gidx_hbm.at[pl.ds(wid * nch, nch)], idxs)
        for b in range(NBUF):
            fire(b, b)

        def group(j0, carry):
            for b in range(NBUF):
                c = j0 * NBUF + b
                wait_fire(c, b)

                @pl.when(j0 > 0)
                def _():
                    wait_out(c - NBUF, b)
                accumulate(b)
                put_out(c, b)

                @pl.when(j0 < nch // NBUF - 1)
                def _():
                    fire(c + NBUF, b)
            return carry

        lax.fori_loop(0, nch // NBUF, group, 0)
        for b in range(NBUF):
            wait_out(nch - NBUF + b, b)

    return functools.partial(
        pl.kernel, mesh=mesh,
        out_type=jax.ShapeDtypeStruct((m, C), jnp.float32),
        scratch_types=scratch,
    )(_body)


def _layernorm(x, g, b):
    mu = jnp.mean(x, axis=-1, keepdims=True)
    var = jnp.mean((x - mu) * (x - mu), axis=-1, keepdims=True)
    return (x - mu) / jnp.sqrt(var + 1e-5) * g + b


def _dense_body(x_ref, t1_ref, s1_ref, wsp_ref, bsp_ref, thl_ref, bl_ref,
                thh_ref, bh_ref, wg1_ref, bg1_ref, g1g_ref, g1b_ref,
                wg2_ref, bg2_ref, wout_ref, bout_ref, og_ref, ob_ref,
                gr_ref, out_ref):
    x = x_ref[...]
    t1 = t1_ref[...]
    t2 = 2.0 * t1 - (2.0 / KNN) * s1_ref[...] - x

    def mm(a, w):
        return jnp.dot(a, w, preferred_element_type=jnp.float32)

    f_sp = mm(x, wsp_ref[...]) + bsp_ref[...]
    f_lo = mm(x, thl_ref[0]) + mm(t1, thl_ref[1]) + mm(t2, thl_ref[2]) + bl_ref[...]
    f_hi = mm(x, thh_ref[0]) + mm(t1, thh_ref[1]) + mm(t2, thh_ref[2]) + bh_ref[...]
    h = (mm(f_sp, wg1_ref[0]) + mm(f_lo, wg1_ref[1]) + mm(f_hi, wg1_ref[2])
         + bg1_ref[...])
    h = jax.nn.relu(_layernorm(h, g1g_ref[...], g1b_ref[...]))
    gate = mm(h, wg2_ref[...]) + bg2_ref[...]
    gate = gate - jnp.max(gate, axis=-1, keepdims=True)
    e = jnp.exp(gate)
    gate = e / jnp.sum(e, axis=-1, keepdims=True)
    f_fused = (gate[:, 0:1] * f_sp + gate[:, 1:2] * f_lo + gate[:, 2:3] * f_hi)
    out = mm(f_fused, wout_ref[...]) + bout_ref[...]
    out = _layernorm(out, og_ref[...], ob_ref[...])
    out_ref[...] = x + gr_ref[0, 0] * out


def _dense_tail(x, t1, s1, wsp, bsp, thl, bl, thh, bh, wg1, bg1, g1g, g1b,
                wg2, bg2, wout, bout, og, ob, gr):
    row = lambda t: (t, 0)
    full2 = lambda t: (0, 0)
    full3 = lambda t: (0, 0, 0)
    specs = [
        pl.BlockSpec((R, C), row),
        pl.BlockSpec((R, C), row),
        pl.BlockSpec((R, C), row),
        pl.BlockSpec((C, C), full2),
        pl.BlockSpec((1, C), full2),
        pl.BlockSpec((3, C, C), full3),
        pl.BlockSpec((1, C), full2),
        pl.BlockSpec((3, C, C), full3),
        pl.BlockSpec((1, C), full2),
        pl.BlockSpec((3, C, C), full3),
        pl.BlockSpec((1, C), full2),
        pl.BlockSpec((1, C), full2),
        pl.BlockSpec((1, C), full2),
        pl.BlockSpec((C, 3), full2),
        pl.BlockSpec((1, 3), full2),
        pl.BlockSpec((C, C), full2),
        pl.BlockSpec((1, C), full2),
        pl.BlockSpec((1, C), full2),
        pl.BlockSpec((1, C), full2),
        pl.BlockSpec((1, 1), full2),
    ]
    m = x.shape[0]
    return pl.pallas_call(
        _dense_body,
        grid=(m // R,),
        in_specs=specs,
        out_specs=pl.BlockSpec((R, C), row),
        out_shape=jax.ShapeDtypeStruct((m, C), jnp.float32),
    )(x, t1, s1, wsp, bsp, thl, bl, thh, bh, wg1, bg1, g1g, g1b,
      wg2, bg2, wout, bout, og, ob, gr)


NSPLIT = 4


@functools.lru_cache(maxsize=None)
def _sc_aggs(m):
    return _make_sc_agg(True, m), _make_sc_agg(False, m)


def kernel(points, feats, W_sp, b_sp, theta_low, b_low, theta_high, b_high,
           W_g1, b_g1, ln_g1_gamma, ln_g1_beta, W_g2, b_g2, W_out, b_out,
           ln_out_gamma, ln_out_beta, gamma_res):
    r2 = lambda v: v.reshape(1, -1)
    nb = B // NSPLIT
    mh = nb * N
    sc_l, sc_sum = _sc_aggs(mh)
    outs = []
    for h in range(NSPLIT):
        pts = points[h * nb:(h + 1) * nb]
        xf = feats[h * nb:(h + 1) * nb].reshape(mh, C)
        gidx = _knn_indices(pts, jnp.swapaxes(pts, 1, 2),
                            nb).reshape(mh // P, P * KNN)
        t1 = sc_l(xf, gidx)
        s1 = sc_sum(t1, gidx)
        out = _dense_tail(
            xf, t1, s1, W_sp, r2(b_sp), theta_low, r2(b_low), theta_high,
            r2(b_high), W_g1.reshape(3, C, C), r2(b_g1), r2(ln_g1_gamma),
            r2(ln_g1_beta), W_g2, r2(b_g2), W_out, r2(b_out),
            r2(ln_out_gamma), r2(ln_out_beta), gamma_res.reshape(1, 1))
        outs.append(out.reshape(nb, N, C))
    return jnp.concatenate(outs, axis=0)


# speedup vs baseline: 1.1015x; 1.0267x over previous
import functools

import jax
import jax.numpy as jnp
from jax import lax
from jax.experimental import pallas as pl
from jax.experimental.pallas import tpu as pltpu
from jax.experimental.pallas import tpu_sc as plsc

B, N, C, KNN = 8, 2048, 128, 16
M = B * N
R = 512
NW = 32
PW = M // NW
P = 8


def _knn_body(pts_ref, ptst_ref, out_ref):
    b = pl.program_id(0)
    r = pl.program_id(1)
    pt = pts_ref[0]
    ptt = ptst_ref[0]
    sq_i = jnp.sum(pt * pt, axis=1, keepdims=True)
    sq_j = jnp.sum(ptt * ptt, axis=0, keepdims=True)
    g = lax.dot_general(pt, ptt, (((1,), (0,)), ((), ())),
                        preferred_element_type=jnp.float32)
    d = sq_i + sq_j - 2.0 * g
    ii = lax.broadcasted_iota(jnp.int32, (R, N), 0) + r * R
    jj = lax.broadcasted_iota(jnp.int32, (R, N), 1)
    d = jnp.where(jj == ii, 1e10, d)
    kb = lax.bitcast_convert_type(d, jnp.int32)
    kb = (kb & jnp.int32(~(N - 1))) | jj
    kf = lax.bitcast_convert_type(kb, jnp.float32)
    cols = []
    for _ in range(KNN):
        m = jnp.min(kf, axis=1, keepdims=True)
        cols.append(lax.bitcast_convert_type(m, jnp.int32) & jnp.int32(N - 1))
        kf = jnp.where(kf == m, jnp.float32(3e38), kf)
    idx = jnp.concatenate(cols, axis=1)
    out_ref[...] = idx + b * N


def _knn_indices(points, pts_t, nb):
    return pl.pallas_call(
        _knn_body,
        grid=(nb, N // R),
        in_specs=[
            pl.BlockSpec((1, R, 3), lambda b, r: (b, r, 0)),
            pl.BlockSpec((1, 3, N), lambda b, r: (b, 0, 0)),
        ],
        out_specs=pl.BlockSpec((R, KNN), lambda b, r: (b * (N // R) + r, 0)),
        out_shape=jax.ShapeDtypeStruct((nb * N, KNN), jnp.int32),
    )(points, pts_t)


NBUF = 4


def _make_sc_agg(with_x, m):
    mesh = plsc.VectorSubcoreMesh(core_axis_name="c", subcore_axis_name="s")
    scale = -1.0 / KNN
    pw = m // NW
    nch = pw // P
    scratch = [pltpu.VMEM((nch, P * KNN), jnp.int32)]
    scratch += [pltpu.VMEM((P * KNN, C), jnp.float32)] * NBUF
    if with_x:
        scratch += [pltpu.VMEM((P, C), jnp.float32)] * NBUF
    scratch += [pltpu.VMEM((P, C), jnp.float32)] * NBUF
    scratch += [pltpu.SemaphoreType.DMA] * (NBUF * (3 if with_x else 2))

    def _body(table_hbm, gidx_hbm, out_hbm, idxs, *bufs):
        rows = bufs[:NBUF]
        k = NBUF
        if with_x:
            xc = bufs[k:k + NBUF]
            k += NBUF
        outb = bufs[k:k + NBUF]
        k += NBUF
        sg = bufs[k:k + NBUF]
        k += NBUF
        if with_x:
            sx = bufs[k:k + NBUF]
            k += NBUF
        so = bufs[k:k + NBUF]

        wid = lax.axis_index("s") * 2 + lax.axis_index("c")
        base = wid * pw

        def fire(c, b):
            pltpu.async_copy(table_hbm.at[idxs.at[c]], rows[b], sg[b])
            if with_x:
                pltpu.async_copy(table_hbm.at[pl.ds(base + c * P, P)],
                                 xc[b], sx[b])

        def wait_fire(c, b):
            pltpu.make_async_copy(table_hbm.at[idxs.at[c]], rows[b],
                                  sg[b]).wait()
            if with_x:
                pltpu.make_async_copy(table_hbm.at[pl.ds(base + c * P, P)],
                                      xc[b], sx[b]).wait()

        def put_out(c, b):
            pltpu.async_copy(outb[b], out_hbm.at[pl.ds(base + c * P, P)],
                             so[b])

        def wait_out(c, b):
            pltpu.make_async_copy(outb[b],
                                  out_hbm.at[pl.ds(base + c * P, P)],
                                  so[b]).wait()

        def accumulate(b):
            def pbody(p, carry):
                for ch in range(C // 16):
                    sl = pl.ds(ch * 16, 16)
                    s = [rows[b][p * KNN + rr, sl]
                         + rows[b][p * KNN + rr + 1, sl]
                         for rr in range(0, KNN, 2)]
                    while len(s) > 1:
                        s = [a + bb for a, bb in zip(s[::2], s[1::2])]
                    if with_x:
                        outb[b][p, sl] = xc[b][p, sl] + scale * s[0]
                    else:
                        outb[b][p, sl] = s[0]
                return carry
            lax.fori_loop(0, P, pbody, 0)

        pltpu.sync_copy(gidx_hbm.at[pl.ds(wid * nch, nch)], idxs)
        for b in range(NBUF):
            fire(b, b)

        def group(j0, carry):
            for b in range(NBUF):
                c = j0 * NBUF + b
                wait_fire(c, b)

                @pl.when(j0 > 0)
                def _():
                    wait_out(c - NBUF, b)
                accumulate(b)
                put_out(c, b)

                @pl.when(j0 < nch // NBUF - 1)
                def _():
                    fire(c + NBUF, b)
            return carry

        lax.fori_loop(0, nch // NBUF, group, 0)
        for b in range(NBUF):
            wait_out(nch - NBUF + b, b)

    return functools.partial(
        pl.kernel, mesh=mesh,
        out_type=jax.ShapeDtypeStruct((m, C), jnp.float32),
        scratch_types=scratch,
    )(_body)


def _layernorm(x, g, b):
    mu = jnp.mean(x, axis=-1, keepdims=True)
    var = jnp.mean((x - mu) * (x - mu), axis=-1, keepdims=True)
    return (x - mu) / jnp.sqrt(var + 1e-5) * g + b


def _dense_body(x_ref, t1_ref, s1_ref, wsp_ref, bsp_ref, thl_ref, bl_ref,
                thh_ref, bh_ref, wg1_ref, bg1_ref, g1g_ref, g1b_ref,
                wg2_ref, bg2_ref, wout_ref, bout_ref, og_ref, ob_ref,
                gr_ref, out_ref):
    x = x_ref[...]
    t1 = t1_ref[...]
    t2 = 2.0 * t1 - (2.0 / KNN) * s1_ref[...] - x

    def mm(a, w):
        return jnp.dot(a, w, preferred_element_type=jnp.float32)

    f_sp = mm(x, wsp_ref[...]) + bsp_ref[...]
    f_lo = mm(x, thl_ref[0]) + mm(t1, thl_ref[1]) + mm(t2, thl_ref[2]) + bl_ref[...]
    f_hi = mm(x, thh_ref[0]) + mm(t1, thh_ref[1]) + mm(t2, thh_ref[2]) + bh_ref[...]
    h = (mm(f_sp, wg1_ref[0]) + mm(f_lo, wg1_ref[1]) + mm(f_hi, wg1_ref[2])
         + bg1_ref[...])
    h = jax.nn.relu(_layernorm(h, g1g_ref[...], g1b_ref[...]))
    gate = mm(h, wg2_ref[...]) + bg2_ref[...]
    gate = gate - jnp.max(gate, axis=-1, keepdims=True)
    e = jnp.exp(gate)
    gate = e / jnp.sum(e, axis=-1, keepdims=True)
    f_fused = (gate[:, 0:1] * f_sp + gate[:, 1:2] * f_lo + gate[:, 2:3] * f_hi)
    out = mm(f_fused, wout_ref[...]) + bout_ref[...]
    out = _layernorm(out, og_ref[...], ob_ref[...])
    out_ref[...] = x + gr_ref[0, 0] * out


def _dense_tail(x, t1, s1, wsp, bsp, thl, bl, thh, bh, wg1, bg1, g1g, g1b,
                wg2, bg2, wout, bout, og, ob, gr):
    row = lambda t: (t, 0)
    full2 = lambda t: (0, 0)
    full3 = lambda t: (0, 0, 0)
    specs = [
        pl.BlockSpec((R, C), row),
        pl.BlockSpec((R, C), row),
        pl.BlockSpec((R, C), row),
        pl.BlockSpec((C, C), full2),
        pl.BlockSpec((1, C), full2),
        pl.BlockSpec((3, C, C), full3),
        pl.BlockSpec((1, C), full2),
        pl.BlockSpec((3, C, C), full3),
        pl.BlockSpec((1, C), full2),
        pl.BlockSpec((3, C, C), full3),
        pl.BlockSpec((1, C), full2),
        pl.BlockSpec((1, C), full2),
        pl.BlockSpec((1, C), full2),
        pl.BlockSpec((C, 3), full2),
        pl.BlockSpec((1, 3), full2),
        pl.BlockSpec((C, C), full2),
        pl.BlockSpec((1, C), full2),
        pl.BlockSpec((1, C), full2),
        pl.BlockSpec((1, C), full2),
        pl.BlockSpec((1, 1), full2),
    ]
    m = x.shape[0]
    return pl.pallas_call(
        _dense_body,
        grid=(m // R,),
        in_specs=specs,
        out_specs=pl.BlockSpec((R, C), row),
        out_shape=jax.ShapeDtypeStruct((m, C), jnp.float32),
    )(x, t1, s1, wsp, bsp, thl, bl, thh, bh, wg1, bg1, g1g, g1b,
      wg2, bg2, wout, bout, og, ob, gr)


NSPLIT = 4


@functools.lru_cache(maxsize=None)
def _sc_aggs(m):
    return _make_sc_agg(True, m), _make_sc_agg(False, m)


def kernel(points, feats, W_sp, b_sp, theta_low, b_low, theta_high, b_high,
           W_g1, b_g1, ln_g1_gamma, ln_g1_beta, W_g2, b_g2, W_out, b_out,
           ln_out_gamma, ln_out_beta, gamma_res):
    r2 = lambda v: v.reshape(1, -1)
    nb = B // NSPLIT
    mh = nb * N
    sc_l, sc_sum = _sc_aggs(mh)
    outs = []
    for h in range(NSPLIT):
        pts = points[h * nb:(h + 1) * nb]
        xf = feats[h * nb:(h + 1) * nb].reshape(mh, C)
        gidx = _knn_indices(pts, jnp.swapaxes(pts, 1, 2),
                            nb).reshape(mh // P, P * KNN)
        t1 = sc_l(xf, gidx)
        s1 = sc_sum(t1, gidx)
        out = _dense_tail(
            xf, t1, s1, W_sp, r2(b_sp), theta_low, r2(b_low), theta_high,
            r2(b_high), W_g1.reshape(3, C, C), r2(b_g1), r2(ln_g1_gamma),
            r2(ln_g1_beta), W_g2, r2(b_g2), W_out, r2(b_out),
            r2(ln_out_gamma), r2(ln_out_beta), gamma_res.reshape(1, 1))
        outs.append(out.reshape(nb, N, C))
    return jnp.concatenate(outs, axis=0)


# speedup vs baseline: 1.1015x; 1.0000x over previous
import functools

import jax
import jax.numpy as jnp
from jax import lax
from jax.experimental import pallas as pl
from jax.experimental.pallas import tpu as pltpu
from jax.experimental.pallas import tpu_sc as plsc

B, N, C, KNN = 8, 2048, 128, 16
M = B * N
R = 512
NW = 32
PW = M // NW
P = 8


def _knn_body(pts_ref, ptst_ref, out_ref):
    b = pl.program_id(0)
    r = pl.program_id(1)
    pt = pts_ref[0]
    ptt = ptst_ref[0]
    sq_i = jnp.sum(pt * pt, axis=1, keepdims=True)
    sq_j = jnp.sum(ptt * ptt, axis=0, keepdims=True)
    g = lax.dot_general(pt, ptt, (((1,), (0,)), ((), ())),
                        preferred_element_type=jnp.float32)
    d = sq_i + sq_j - 2.0 * g
    ii = lax.broadcasted_iota(jnp.int32, (R, N), 0) + r * R
    jj = lax.broadcasted_iota(jnp.int32, (R, N), 1)
    d = jnp.where(jj == ii, 1e10, d)
    kb = lax.bitcast_convert_type(d, jnp.int32)
    kb = (kb & jnp.int32(~(N - 1))) | jj
    kf = lax.bitcast_convert_type(kb, jnp.float32)
    cols = []
    for t in range(KNN):
        m = jnp.min(kf, axis=1, keepdims=True)
        cols.append(lax.bitcast_convert_type(m, jnp.int32) & jnp.int32(N - 1))
        if t < KNN - 1:
            kf = jnp.where(kf == m, jnp.float32(3e38), kf)
    idx = jnp.concatenate(cols, axis=1)
    out_ref[...] = idx + b * N


def _knn_indices(points, pts_t, nb):
    return pl.pallas_call(
        _knn_body,
        grid=(nb, N // R),
        in_specs=[
            pl.BlockSpec((1, R, 3), lambda b, r: (b, r, 0)),
            pl.BlockSpec((1, 3, N), lambda b, r: (b, 0, 0)),
        ],
        out_specs=pl.BlockSpec((R, KNN), lambda b, r: (b * (N // R) + r, 0)),
        out_shape=jax.ShapeDtypeStruct((nb * N, KNN), jnp.int32),
    )(points, pts_t)


NBUF = 4


def _make_sc_agg(with_x, m):
    mesh = plsc.VectorSubcoreMesh(core_axis_name="c", subcore_axis_name="s")
    scale = -1.0 / KNN
    pw = m // NW
    nch = pw // P
    scratch = [pltpu.VMEM((nch, P * KNN), jnp.int32)]
    scratch += [pltpu.VMEM((P * KNN, C), jnp.float32)] * NBUF
    if with_x:
        scratch += [pltpu.VMEM((P, C), jnp.float32)] * NBUF
    scratch += [pltpu.VMEM((P, C), jnp.float32)] * NBUF
    scratch += [pltpu.SemaphoreType.DMA] * (NBUF * (3 if with_x else 2))

    def _body(table_hbm, gidx_hbm, out_hbm, idxs, *bufs):
        rows = bufs[:NBUF]
        k = NBUF
        if with_x:
            xc = bufs[k:k + NBUF]
            k += NBUF
        outb = bufs[k:k + NBUF]
        k += NBUF
        sg = bufs[k:k + NBUF]
        k += NBUF
        if with_x:
            sx = bufs[k:k + NBUF]
            k += NBUF
        so = bufs[k:k + NBUF]

        wid = lax.axis_index("s") * 2 + lax.axis_index("c")
        base = wid * pw

        def fire(c, b):
            pltpu.async_copy(table_hbm.at[idxs.at[c]], rows[b], sg[b])
            if with_x:
                pltpu.async_copy(table_hbm.at[pl.ds(base + c * P, P)],
                                 xc[b], sx[b])

        def wait_fire(c, b):
            pltpu.make_async_copy(table_hbm.at[idxs.at[c]], rows[b],
                                  sg[b]).wait()
            if with_x:
                pltpu.make_async_copy(table_hbm.at[pl.ds(base + c * P, P)],
                                      xc[b], sx[b]).wait()

        def put_out(c, b):
            pltpu.async_copy(outb[b], out_hbm.at[pl.ds(base + c * P, P)],
                             so[b])

        def wait_out(c, b):
            pltpu.make_async_copy(outb[b],
                                  out_hbm.at[pl.ds(base + c * P, P)],
                                  so[b]).wait()

        def accumulate(b):
            def pbody(p, carry):
                for ch in range(C // 16):
                    sl = pl.ds(ch * 16, 16)
                    s = [rows[b][p * KNN + rr, sl]
                         + rows[b][p * KNN + rr + 1, sl]
                         for rr in range(0, KNN, 2)]
                    while len(s) > 1:
                        s = [a + bb for a, bb in zip(s[::2], s[1::2])]
                    if with_x:
                        outb[b][p, sl] = xc[b][p, sl] + scale * s[0]
                    else:
                        outb[b][p, sl] = s[0]
                return carry
            lax.fori_loop(0, P, pbody, 0)

        pltpu.sync_copy(gidx_hbm.at[pl.ds(wid * nch, nch)], idxs)
        for b in range(NBUF):
            fire(b, b)

        def group(j0, carry):
            for b in range(NBUF):
                c = j0 * NBUF + b
                wait_fire(c, b)

                @pl.when(j0 > 0)
                def _():
                    wait_out(c - NBUF, b)
                accumulate(b)
                put_out(c, b)

                @pl.when(j0 < nch // NBUF - 1)
                def _():
                    fire(c + NBUF, b)
            return carry

        lax.fori_loop(0, nch // NBUF, group, 0)
        for b in range(NBUF):
            wait_out(nch - NBUF + b, b)

    return functools.partial(
        pl.kernel, mesh=mesh,
        out_type=jax.ShapeDtypeStruct((m, C), jnp.float32),
        scratch_types=scratch,
    )(_body)


def _layernorm(x, g, b):
    mu = jnp.mean(x, axis=-1, keepdims=True)
    var = jnp.mean((x - mu) * (x - mu), axis=-1, keepdims=True)
    return (x - mu) / jnp.sqrt(var + 1e-5) * g + b


def _dense_body(x_ref, t1_ref, s1_ref, wsp_ref, bsp_ref, thl_ref, bl_ref,
                thh_ref, bh_ref, wg1_ref, bg1_ref, g1g_ref, g1b_ref,
                wg2_ref, bg2_ref, wout_ref, bout_ref, og_ref, ob_ref,
                gr_ref, out_ref):
    x = x_ref[...]
    t1 = t1_ref[...]
    t2 = 2.0 * t1 - (2.0 / KNN) * s1_ref[...] - x

    def mm(a, w):
        return jnp.dot(a, w, preferred_element_type=jnp.float32)

    f_sp = mm(x, wsp_ref[...]) + bsp_ref[...]
    f_lo = mm(x, thl_ref[0]) + mm(t1, thl_ref[1]) + mm(t2, thl_ref[2]) + bl_ref[...]
    f_hi = mm(x, thh_ref[0]) + mm(t1, thh_ref[1]) + mm(t2, thh_ref[2]) + bh_ref[...]
    h = (mm(f_sp, wg1_ref[0]) + mm(f_lo, wg1_ref[1]) + mm(f_hi, wg1_ref[2])
         + bg1_ref[...])
    h = jax.nn.relu(_layernorm(h, g1g_ref[...], g1b_ref[...]))
    gate = mm(h, wg2_ref[...]) + bg2_ref[...]
    gate = gate - jnp.max(gate, axis=-1, keepdims=True)
    e = jnp.exp(gate)
    gate = e / jnp.sum(e, axis=-1, keepdims=True)
    f_fused = (gate[:, 0:1] * f_sp + gate[:, 1:2] * f_lo + gate[:, 2:3] * f_hi)
    out = mm(f_fused, wout_ref[...]) + bout_ref[...]
    out = _layernorm(out, og_ref[...], ob_ref[...])
    out_ref[...] = x + gr_ref[0, 0] * out


def _dense_tail(x, t1, s1, wsp, bsp, thl, bl, thh, bh, wg1, bg1, g1g, g1b,
                wg2, bg2, wout, bout, og, ob, gr):
    row = lambda t: (t, 0)
    full2 = lambda t: (0, 0)
    full3 = lambda t: (0, 0, 0)
    specs = [
        pl.BlockSpec((R, C), row),
        pl.BlockSpec((R, C), row),
        pl.BlockSpec((R, C), row),
        pl.BlockSpec((C, C), full2),
        pl.BlockSpec((1, C), full2),
        pl.BlockSpec((3, C, C), full3),
        pl.BlockSpec((1, C), full2),
        pl.BlockSpec((3, C, C), full3),
        pl.BlockSpec((1, C), full2),
        pl.BlockSpec((3, C, C), full3),
        pl.BlockSpec((1, C), full2),
        pl.BlockSpec((1, C), full2),
        pl.BlockSpec((1, C), full2),
        pl.BlockSpec((C, 3), full2),
        pl.BlockSpec((1, 3), full2),
        pl.BlockSpec((C, C), full2),
        pl.BlockSpec((1, C), full2),
        pl.BlockSpec((1, C), full2),
        pl.BlockSpec((1, C), full2),
        pl.BlockSpec((1, 1), full2),
    ]
    m = x.shape[0]
    return pl.pallas_call(
        _dense_body,
        grid=(m // R,),
        in_specs=specs,
        out_specs=pl.BlockSpec((R, C), row),
        out_shape=jax.ShapeDtypeStruct((m, C), jnp.float32),
    )(x, t1, s1, wsp, bsp, thl, bl, thh, bh, wg1, bg1, g1g, g1b,
      wg2, bg2, wout, bout, og, ob, gr)


NSPLIT = 4


@functools.lru_cache(maxsize=None)
def _sc_aggs(m):
    return _make_sc_agg(True, m), _make_sc_agg(False, m)


def kernel(points, feats, W_sp, b_sp, theta_low, b_low, theta_high, b_high,
           W_g1, b_g1, ln_g1_gamma, ln_g1_beta, W_g2, b_g2, W_out, b_out,
           ln_out_gamma, ln_out_beta, gamma_res):
    r2 = lambda v: v.reshape(1, -1)
    nb = B // NSPLIT
    mh = nb * N
    sc_l, sc_sum = _sc_aggs(mh)
    outs = []
    for h in range(NSPLIT):
        pts = points[h * nb:(h + 1) * nb]
        xf = feats[h * nb:(h + 1) * nb].reshape(mh, C)
        gidx = _knn_indices(pts, jnp.swapaxes(pts, 1, 2),
                            nb).reshape(mh // P, P * KNN)
        t1 = sc_l(xf, gidx)
        s1 = sc_sum(t1, gidx)
        out = _dense_tail(
            xf, t1, s1, W_sp, r2(b_sp), theta_low, r2(b_low), theta_high,
            r2(b_high), W_g1.reshape(3, C, C), r2(b_g1), r2(ln_g1_gamma),
            r2(ln_g1_beta), W_g2, r2(b_g2), W_out, r2(b_out),
            r2(ln_out_gamma), r2(ln_out_beta), gamma_res.reshape(1, 1))
        outs.append(out.reshape(nb, N, C))
    return jnp.concatenate(outs, axis=0)


# speedup vs baseline: 1.1023x; 1.0007x over previous
import functools

import jax
import jax.numpy as jnp
from jax import lax
from jax.experimental import pallas as pl
from jax.experimental.pallas import tpu as pltpu
from jax.experimental.pallas import tpu_sc as plsc

B, N, C, KNN = 8, 2048, 128, 16
M = B * N
R = 512
NW = 32
PW = M // NW
P = 8


def _knn_body(pts_ref, ptst_ref, out_ref):
    b = pl.program_id(0)
    r = pl.program_id(1)
    pt = pts_ref[0]
    ptt = ptst_ref[0]
    sq_i = jnp.sum(pt * pt, axis=1, keepdims=True)
    sq_j = jnp.sum(ptt * ptt, axis=0, keepdims=True)
    g = lax.dot_general(pt, ptt, (((1,), (0,)), ((), ())),
                        preferred_element_type=jnp.float32)
    d = sq_i + sq_j - 2.0 * g
    ii = lax.broadcasted_iota(jnp.int32, (R, N), 0) + r * R
    jj = lax.broadcasted_iota(jnp.int32, (R, N), 1)
    d = jnp.where(jj == ii, 1e10, d)
    kb = lax.bitcast_convert_type(d, jnp.int32)
    kb = (kb & jnp.int32(~(N - 1))) | jj
    kf = lax.bitcast_convert_type(kb, jnp.float32)
    cols = []
    for t in range(KNN):
        m = jnp.min(kf, axis=1, keepdims=True)
        cols.append(lax.bitcast_convert_type(m, jnp.int32) & jnp.int32(N - 1))
        if t < KNN - 1:
            kf = jnp.where(kf == m, jnp.float32(3e38), kf)
    idx = jnp.concatenate(cols, axis=1)
    out_ref[...] = idx + b * N


def _knn_indices(points, pts_t, nb):
    return pl.pallas_call(
        _knn_body,
        grid=(nb, N // R),
        in_specs=[
            pl.BlockSpec((1, R, 3), lambda b, r: (b, r, 0)),
            pl.BlockSpec((1, 3, N), lambda b, r: (b, 0, 0)),
        ],
        out_specs=pl.BlockSpec((R, KNN), lambda b, r: (b * (N // R) + r, 0)),
        out_shape=jax.ShapeDtypeStruct((nb * N, KNN), jnp.int32),
    )(points, pts_t)


NBUF = 4


def _make_sc_agg(with_x, m):
    mesh = plsc.VectorSubcoreMesh(core_axis_name="c", subcore_axis_name="s")
    scale = -1.0 / KNN
    pw = m // NW
    nch = pw // P
    scratch = [pltpu.VMEM((nch, P * KNN), jnp.int32)]
    scratch += [pltpu.VMEM((P * KNN, C), jnp.float32)] * NBUF
    if with_x:
        scratch += [pltpu.VMEM((P, C), jnp.float32)] * NBUF
    scratch += [pltpu.VMEM((P, C), jnp.float32)] * NBUF
    scratch += [pltpu.SemaphoreType.DMA] * (NBUF * (3 if with_x else 2))

    def _body(table_hbm, gidx_hbm, out_hbm, idxs, *bufs):
        rows = bufs[:NBUF]
        k = NBUF
        if with_x:
            xc = bufs[k:k + NBUF]
            k += NBUF
        outb = bufs[k:k + NBUF]
        k += NBUF
        sg = bufs[k:k + NBUF]
        k += NBUF
        if with_x:
            sx = bufs[k:k + NBUF]
            k += NBUF
        so = bufs[k:k + NBUF]

        wid = lax.axis_index("s") * 2 + lax.axis_index("c")
        base = wid * pw

        def fire(c, b):
            pltpu.async_copy(table_hbm.at[idxs.at[c]], rows[b], sg[b])
            if with_x:
                pltpu.async_copy(table_hbm.at[pl.ds(base + c * P, P)],
                                 xc[b], sx[b])

        def wait_fire(c, b):
            pltpu.make_async_copy(table_hbm.at[idxs.at[c]], rows[b],
                                  sg[b]).wait()
            if with_x:
                pltpu.make_async_copy(table_hbm.at[pl.ds(base + c * P, P)],
                                      xc[b], sx[b]).wait()

        def put_out(c, b):
            pltpu.async_copy(outb[b], out_hbm.at[pl.ds(base + c * P, P)],
                             so[b])

        def wait_out(c, b):
            pltpu.make_async_copy(outb[b],
                                  out_hbm.at[pl.ds(base + c * P, P)],
                                  so[b]).wait()

        def accumulate(b):
            def pbody(p, carry):
                for ch in range(C // 16):
                    sl = pl.ds(ch * 16, 16)
                    s = [rows[b][p * KNN + rr, sl]
                         + rows[b][p * KNN + rr + 1, sl]
                         for rr in range(0, KNN, 2)]
                    while len(s) > 1:
                        s = [a + bb for a, bb in zip(s[::2], s[1::2])]
                    if with_x:
                        outb[b][p, sl] = xc[b][p, sl] + scale * s[0]
                    else:
                        outb[b][p, sl] = s[0]
                return carry
            lax.fori_loop(0, P, pbody, 0)

        pltpu.sync_copy(gidx_hbm.at[pl.ds(wid * nch, nch)], idxs)
        for b in range(NBUF):
            fire(b, b)

        def group(j0, carry):
            for b in range(NBUF):
                c = j0 * NBUF + b
                wait_fire(c, b)

                @pl.when(j0 > 0)
                def _():
                    wait_out(c - NBUF, b)
                accumulate(b)
                put_out(c, b)

                @pl.when(j0 < nch // NBUF - 1)
                def _():
                    fire(c + NBUF, b)
            return carry

        lax.fori_loop(0, nch // NBUF, group, 0)
        for b in range(NBUF):
            wait_out(nch - NBUF + b, b)

    return functools.partial(
        pl.kernel, mesh=mesh,
        out_type=jax.ShapeDtypeStruct((m, C), jnp.float32),
        scratch_types=scratch,
    )(_body)


def _layernorm(x, g, b):
    mu = jnp.mean(x, axis=-1, keepdims=True)
    var = jnp.mean((x - mu) * (x - mu), axis=-1, keepdims=True)
    return (x - mu) / jnp.sqrt(var + 1e-5) * g + b


def _dense_body(x_ref, t1_ref, s1_ref, wsp_ref, bsp_ref, thl_ref, bl_ref,
                thh_ref, bh_ref, wg1_ref, bg1_ref, g1g_ref, g1b_ref,
                wg2_ref, bg2_ref, wout_ref, bout_ref, og_ref, ob_ref,
                gr_ref, out_ref):
    x = x_ref[...]
    t1 = t1_ref[...]
    t2 = 2.0 * t1 - (2.0 / KNN) * s1_ref[...] - x

    def mm(a, w):
        return jnp.dot(a.astype(jnp.bfloat16), w.astype(jnp.bfloat16),
                       preferred_element_type=jnp.float32)

    f_sp = mm(x, wsp_ref[...]) + bsp_ref[...]
    f_lo = mm(x, thl_ref[0]) + mm(t1, thl_ref[1]) + mm(t2, thl_ref[2]) + bl_ref[...]
    f_hi = mm(x, thh_ref[0]) + mm(t1, thh_ref[1]) + mm(t2, thh_ref[2]) + bh_ref[...]
    h = (mm(f_sp, wg1_ref[0]) + mm(f_lo, wg1_ref[1]) + mm(f_hi, wg1_ref[2])
         + bg1_ref[...])
    h = jax.nn.relu(_layernorm(h, g1g_ref[...], g1b_ref[...]))
    gate = mm(h, wg2_ref[...]) + bg2_ref[...]
    gate = gate - jnp.max(gate, axis=-1, keepdims=True)
    e = jnp.exp(gate)
    gate = e / jnp.sum(e, axis=-1, keepdims=True)
    f_fused = (gate[:, 0:1] * f_sp + gate[:, 1:2] * f_lo + gate[:, 2:3] * f_hi)
    out = mm(f_fused, wout_ref[...]) + bout_ref[...]
    out = _layernorm(out, og_ref[...], ob_ref[...])
    out_ref[...] = x + gr_ref[0, 0] * out


def _dense_tail(x, t1, s1, wsp, bsp, thl, bl, thh, bh, wg1, bg1, g1g, g1b,
                wg2, bg2, wout, bout, og, ob, gr):
    row = lambda t: (t, 0)
    full2 = lambda t: (0, 0)
    full3 = lambda t: (0, 0, 0)
    specs = [
        pl.BlockSpec((R, C), row),
        pl.BlockSpec((R, C), row),
        pl.BlockSpec((R, C), row),
        pl.BlockSpec((C, C), full2),
        pl.BlockSpec((1, C), full2),
        pl.BlockSpec((3, C, C), full3),
        pl.BlockSpec((1, C), full2),
        pl.BlockSpec((3, C, C), full3),
        pl.BlockSpec((1, C), full2),
        pl.BlockSpec((3, C, C), full3),
        pl.BlockSpec((1, C), full2),
        pl.BlockSpec((1, C), full2),
        pl.BlockSpec((1, C), full2),
        pl.BlockSpec((C, 3), full2),
        pl.BlockSpec((1, 3), full2),
        pl.BlockSpec((C, C), full2),
        pl.BlockSpec((1, C), full2),
        pl.BlockSpec((1, C), full2),
        pl.BlockSpec((1, C), full2),
        pl.BlockSpec((1, 1), full2),
    ]
    m = x.shape[0]
    return pl.pallas_call(
        _dense_body,
        grid=(m // R,),
        in_specs=specs,
        out_specs=pl.BlockSpec((R, C), row),
        out_shape=jax.ShapeDtypeStruct((m, C), jnp.float32),
    )(x, t1, s1, wsp, bsp, thl, bl, thh, bh, wg1, bg1, g1g, g1b,
      wg2, bg2, wout, bout, og, ob, gr)


NSPLIT = 4


@functools.lru_cache(maxsize=None)
def _sc_aggs(m):
    return _make_sc_agg(True, m), _make_sc_agg(False, m)


def kernel(points, feats, W_sp, b_sp, theta_low, b_low, theta_high, b_high,
           W_g1, b_g1, ln_g1_gamma, ln_g1_beta, W_g2, b_g2, W_out, b_out,
           ln_out_gamma, ln_out_beta, gamma_res):
    r2 = lambda v: v.reshape(1, -1)
    nb = B // NSPLIT
    mh = nb * N
    sc_l, sc_sum = _sc_aggs(mh)
    outs = []
    for h in range(NSPLIT):
        pts = points[h * nb:(h + 1) * nb]
        xf = feats[h * nb:(h + 1) * nb].reshape(mh, C)
        gidx = _knn_indices(pts, jnp.swapaxes(pts, 1, 2),
                            nb).reshape(mh // P, P * KNN)
        t1 = sc_l(xf, gidx)
        s1 = sc_sum(t1, gidx)
        out = _dense_tail(
            xf, t1, s1, W_sp, r2(b_sp), theta_low, r2(b_low), theta_high,
            r2(b_high), W_g1.reshape(3, C, C), r2(b_g1), r2(ln_g1_gamma),
            r2(ln_g1_beta), W_g2, r2(b_g2), W_out, r2(b_out),
            r2(ln_out_gamma), r2(ln_out_beta), gamma_res.reshape(1, 1))
        outs.append(out.reshape(nb, N, C))
    return jnp.concatenate(outs, axis=0)
